# Initial kernel scaffold; baseline (speedup 1.0000x reference)
import jax
import jax.numpy as jnp
from jax.experimental import pallas as pl


def kernel(x, emb_tables, W1, b1, W2, b2, W3, b3):
    raise NotImplementedError("write your pallas kernel here")


# speedup vs baseline: 7.5113x; 7.5113x over previous
import functools

import jax
import jax.numpy as jnp
from jax import lax
from jax.experimental import pallas as pl
from jax.experimental.pallas import tpu as pltpu
from jax.experimental.pallas import tpu_sc as plsc

B = 16384
NV = 26
CARD = 100000
ED = 16
IN_DIM = NV * ED
H1 = 128
H2 = 64

NC = 2
NS = 16
LANES = 16
NW = NC * NS
BN = B * NV
NWR = BN // NW
GROUP = 128
NG = NWR // GROUP
CHUNKS = GROUP // LANES


def _sc_gather_body(table_hbm, x_hbm, out_hbm, x_v, idx_v, rows_v, sem0, sem1):
    wid = lax.axis_index("s") * NC + lax.axis_index("c")
    row0 = wid * (NWR // GROUP)
    base_flat = wid * NWR

    pltpu.sync_copy(x_hbm.at[pl.ds(row0, NG)], x_v)

    lane = lax.broadcasted_iota(jnp.int32, (LANES,), 0)

    def adjust_row(r, _):
        for k in range(CHUNKS):
            pos = lane + (r * GROUP + k * LANES)
            off = lax.rem(pos, NV) * CARD
            idx_v[r, pl.ds(k * LANES, LANES)] = x_v[r, pl.ds(k * LANES, LANES)] + off
        return 0

    lax.fori_loop(0, NG, adjust_row, 0)

    def gather(g, buf_ref, sem):
        return pltpu.make_async_copy(table_hbm.at[idx_v.at[g]], buf_ref, sem)

    def write(g, buf_ref):
        pltpu.sync_copy(buf_ref, out_hbm.at[pl.ds(base_flat + g * GROUP, GROUP)])

    gather(0, rows_v.at[0], sem0).start()

    def pair(p, _):
        g0 = p * 2
        gather(g0, rows_v.at[0], sem0).wait()
        gather(g0 + 1, rows_v.at[1], sem1).start()
        write(g0, rows_v.at[0])
        gather(g0 + 1, rows_v.at[1], sem1).wait()

        @pl.when(p < NG // 2 - 1)
        def _():
            gather(g0 + 2, rows_v.at[0], sem0).start()

        write(g0 + 1, rows_v.at[1])
        return 0

    lax.fori_loop(0, NG // 2, pair, 0)


def _sc_gather(table, x2d):
    mesh = plsc.VectorSubcoreMesh(
        core_axis_name="c", subcore_axis_name="s", num_cores=NC, num_subcores=NS
    )
    fn = pl.kernel(
        _sc_gather_body,
        out_type=jax.ShapeDtypeStruct((BN, ED), jnp.float32),
        mesh=mesh,
        compiler_params=pltpu.CompilerParams(use_tc_tiling_on_sc=False),
        scratch_types=[
            pltpu.VMEM((NG, GROUP), jnp.int32),
            pltpu.VMEM((NG, GROUP), jnp.int32),
            pltpu.VMEM((2, GROUP, ED), jnp.float32),
            pltpu.SemaphoreType.DMA,
            pltpu.SemaphoreType.DMA,
        ],
    )
    return fn(table, x2d)


def _mlp_body(enc_ref, w1_ref, b1_ref, w2_ref, b2_ref, w3_ref, b3_ref, out_ref):
    h = jnp.dot(enc_ref[...], w1_ref[...], preferred_element_type=jnp.float32)
    h = jnp.maximum(h + b1_ref[...], 0.0)
    h = jnp.dot(h, w2_ref[...], preferred_element_type=jnp.float32)
    h = jnp.maximum(h + b2_ref[...], 0.0)
    out_ref[...] = jnp.dot(h, w3_ref[...], preferred_element_type=jnp.float32) + b3_ref[...]


def _tc_mlp(enc, W1, b1, W2, b2, W3, b3):
    BB = 2048
    grid = (B // BB,)
    return pl.pallas_call(
        _mlp_body,
        grid=grid,
        in_specs=[
            pl.BlockSpec((BB, IN_DIM), lambda i: (i, 0)),
            pl.BlockSpec((IN_DIM, H1), lambda i: (0, 0)),
            pl.BlockSpec((1, H1), lambda i: (0, 0)),
            pl.BlockSpec((H1, H2), lambda i: (0, 0)),
            pl.BlockSpec((1, H2), lambda i: (0, 0)),
            pl.BlockSpec((H2, 1), lambda i: (0, 0)),
            pl.BlockSpec((1, 1), lambda i: (0, 0)),
        ],
        out_specs=pl.BlockSpec((BB, 1), lambda i: (i, 0)),
        out_shape=jax.ShapeDtypeStruct((B, 1), jnp.float32),
    )(enc, W1, b1, W2, b2, W3, b3)


def kernel(x, emb_tables, W1, b1, W2, b2, W3, b3):
    table = emb_tables.reshape(NV * CARD, ED)
    x2d = x.reshape(BN // GROUP, GROUP)
    enc = _sc_gather(table, x2d).reshape(B, IN_DIM)
    return _tc_mlp(
        enc, W1, b1.reshape(1, H1), W2, b2.reshape(1, H2), W3, b3.reshape(1, 1)
    )


# speedup vs baseline: 14.6617x; 1.9520x over previous
import functools

import jax
import jax.numpy as jnp
from jax import lax
from jax.experimental import pallas as pl
from jax.experimental.pallas import tpu as pltpu
from jax.experimental.pallas import tpu_sc as plsc

B = 16384
NV = 26
CARD = 100000
ED = 16
IN_DIM = NV * ED
H1 = 128
H2 = 64

NC = 2
NS = 16
NW = NC * NS
BW = B // NW
GROUP = 128
CPF = BW // GROUP
NG = NV * CPF


def _sc_gather_body(table_hbm, xt_hbm, out_hbm, idx_v, rows_v, sem0, sem1):
    wid = lax.axis_index("s") * NC + lax.axis_index("c")
    b0 = wid * BW

    pltpu.sync_copy(xt_hbm.at[:, pl.ds(wid * CPF, CPF)], idx_v)

    def fire(v, c, buf_ref, sem):
        for e in range(ED):
            pltpu.make_async_copy(
                table_hbm.at[v, e].at[idx_v.at[v, c]], buf_ref.at[e], sem
            ).start()

    def drain(v, c, buf_ref, sem):
        for e in range(ED):
            pltpu.make_async_copy(
                table_hbm.at[v, e].at[idx_v.at[v, c]], buf_ref.at[e], sem
            ).wait()

    def write(v, c, buf_ref):
        pltpu.sync_copy(
            buf_ref, out_hbm.at[v, :, pl.ds(b0 + c * GROUP, GROUP)]
        )

    fire(0, 0, rows_v.at[0], sem0)

    def pair(p, _):
        g0 = p * 2
        v0, c0 = lax.div(g0, CPF), lax.rem(g0, CPF)
        v1, c1 = lax.div(g0 + 1, CPF), lax.rem(g0 + 1, CPF)
        drain(v0, c0, rows_v.at[0], sem0)
        fire(v1, c1, rows_v.at[1], sem1)
        write(v0, c0, rows_v.at[0])
        drain(v1, c1, rows_v.at[1], sem1)

        @pl.when(p < NG // 2 - 1)
        def _():
            v2, c2 = lax.div(g0 + 2, CPF), lax.rem(g0 + 2, CPF)
            fire(v2, c2, rows_v.at[0], sem0)

        write(v1, c1, rows_v.at[1])
        return 0

    lax.fori_loop(0, NG // 2, pair, 0)


def _sc_gather(tableT, xt3):
    mesh = plsc.VectorSubcoreMesh(
        core_axis_name="c", subcore_axis_name="s", num_cores=NC, num_subcores=NS
    )
    fn = pl.kernel(
        _sc_gather_body,
        out_type=jax.ShapeDtypeStruct((NV, ED, B), jnp.float32),
        mesh=mesh,
        compiler_params=pltpu.CompilerParams(use_tc_tiling_on_sc=False),
        scratch_types=[
            pltpu.VMEM((NV, CPF, GROUP), jnp.int32),
            pltpu.VMEM((2, ED, GROUP), jnp.float32),
            pltpu.SemaphoreType.DMA,
            pltpu.SemaphoreType.DMA,
        ],
    )
    return fn(tableT, xt3)


def _mlp_body(enc_ref, w1_ref, b1_ref, w2_ref, b2_ref, w3_ref, b3_ref, out_ref):
    h = jnp.dot(w1_ref[...], enc_ref[...], preferred_element_type=jnp.float32)
    h = jnp.maximum(h + b1_ref[...], 0.0)
    h = jnp.dot(w2_ref[...], h, preferred_element_type=jnp.float32)
    h = jnp.maximum(h + b2_ref[...], 0.0)
    out_ref[...] = jnp.dot(w3_ref[...], h, preferred_element_type=jnp.float32) + b3_ref[...]


def _tc_mlp_t(encT, W1t, b1, W2t, b2, W3t, b3):
    BB = 2048
    grid = (B // BB,)
    return pl.pallas_call(
        _mlp_body,
        grid=grid,
        in_specs=[
            pl.BlockSpec((IN_DIM, BB), lambda i: (0, i)),
            pl.BlockSpec((H1, IN_DIM), lambda i: (0, 0)),
            pl.BlockSpec((H1, 1), lambda i: (0, 0)),
            pl.BlockSpec((H2, H1), lambda i: (0, 0)),
            pl.BlockSpec((H2, 1), lambda i: (0, 0)),
            pl.BlockSpec((1, H2), lambda i: (0, 0)),
            pl.BlockSpec((1, 1), lambda i: (0, 0)),
        ],
        out_specs=pl.BlockSpec((1, BB), lambda i: (0, i)),
        out_shape=jax.ShapeDtypeStruct((1, B), jnp.float32),
    )(encT, W1t, b1, W2t, b2, W3t, b3)


def kernel(x, emb_tables, W1, b1, W2, b2, W3, b3):
    tableT = emb_tables.transpose(0, 2, 1)
    xt3 = x.T.reshape(NV, B // GROUP, GROUP)
    encT = _sc_gather(tableT, xt3).reshape(IN_DIM, B)
    outT = _tc_mlp_t(
        encT,
        W1.T, b1.reshape(H1, 1),
        W2.T, b2.reshape(H2, 1),
        W3.T, b3.reshape(1, 1),
    )
    return outT.reshape(B, 1)


# speedup vs baseline: 16.0381x; 1.0939x over previous
import functools

import jax
import jax.numpy as jnp
from jax import lax
from jax.experimental import pallas as pl
from jax.experimental.pallas import tpu as pltpu
from jax.experimental.pallas import tpu_sc as plsc

B = 16384
NV = 26
CARD = 100000
ED = 16
IN_DIM = NV * ED
H1 = 128
H2 = 64

NC = 2
NS = 16
NW = NC * NS
BW = B // NW
GROUP = 512
CPF = BW // GROUP
NG = NV * CPF


def _sc_gather_body(table_hbm, xt_hbm, out_hbm, idx_v, rows_v, sem0, sem1):
    wid = lax.axis_index("s") * NC + lax.axis_index("c")
    b0 = wid * BW

    pltpu.sync_copy(xt_hbm.at[:, pl.ds(wid * CPF, CPF)], idx_v)

    def fire(v, c, buf_ref, sem):
        for e in range(ED):
            pltpu.make_async_copy(
                table_hbm.at[v, e].at[idx_v.at[v, c]], buf_ref.at[e], sem
            ).start()

    def drain(v, c, buf_ref, sem):
        for e in range(ED):
            pltpu.make_async_copy(
                table_hbm.at[v, e].at[idx_v.at[v, c]], buf_ref.at[e], sem
            ).wait()

    def write(v, c, buf_ref):
        pltpu.sync_copy(
            buf_ref, out_hbm.at[v, :, pl.ds(b0 + c * GROUP, GROUP)]
        )

    fire(0, 0, rows_v.at[0], sem0)

    def pair(p, _):
        g0 = p * 2
        v0, c0 = lax.div(g0, CPF), lax.rem(g0, CPF)
        v1, c1 = lax.div(g0 + 1, CPF), lax.rem(g0 + 1, CPF)
        drain(v0, c0, rows_v.at[0], sem0)
        fire(v1, c1, rows_v.at[1], sem1)
        write(v0, c0, rows_v.at[0])
        drain(v1, c1, rows_v.at[1], sem1)

        @pl.when(p < NG // 2 - 1)
        def _():
            v2, c2 = lax.div(g0 + 2, CPF), lax.rem(g0 + 2, CPF)
            fire(v2, c2, rows_v.at[0], sem0)

        write(v1, c1, rows_v.at[1])
        return 0

    lax.fori_loop(0, NG // 2, pair, 0)


def _sc_gather(tableT, xt3):
    mesh = plsc.VectorSubcoreMesh(
        core_axis_name="c", subcore_axis_name="s", num_cores=NC, num_subcores=NS
    )
    fn = pl.kernel(
        _sc_gather_body,
        out_type=jax.ShapeDtypeStruct((NV, ED, B), jnp.float32),
        mesh=mesh,
        compiler_params=pltpu.CompilerParams(use_tc_tiling_on_sc=False),
        scratch_types=[
            pltpu.VMEM((NV, CPF, GROUP), jnp.int32),
            pltpu.VMEM((2, ED, GROUP), jnp.float32),
            pltpu.SemaphoreType.DMA,
            pltpu.SemaphoreType.DMA,
        ],
    )
    return fn(tableT, xt3)


def _mlp_body(enc_ref, w1_ref, b1_ref, w2_ref, b2_ref, w3_ref, b3_ref, out_ref):
    h = jnp.dot(w1_ref[...], enc_ref[...], preferred_element_type=jnp.float32)
    h = jnp.maximum(h + b1_ref[...], 0.0)
    h = jnp.dot(w2_ref[...], h, preferred_element_type=jnp.float32)
    h = jnp.maximum(h + b2_ref[...], 0.0)
    out_ref[...] = jnp.dot(w3_ref[...], h, preferred_element_type=jnp.float32) + b3_ref[...]


def _tc_mlp_t(encT, W1t, b1, W2t, b2, W3t, b3):
    BB = 2048
    grid = (B // BB,)
    return pl.pallas_call(
        _mlp_body,
        grid=grid,
        in_specs=[
            pl.BlockSpec((IN_DIM, BB), lambda i: (0, i)),
            pl.BlockSpec((H1, IN_DIM), lambda i: (0, 0)),
            pl.BlockSpec((H1, 1), lambda i: (0, 0)),
            pl.BlockSpec((H2, H1), lambda i: (0, 0)),
            pl.BlockSpec((H2, 1), lambda i: (0, 0)),
            pl.BlockSpec((1, H2), lambda i: (0, 0)),
            pl.BlockSpec((1, 1), lambda i: (0, 0)),
        ],
        out_specs=pl.BlockSpec((1, BB), lambda i: (0, i)),
        out_shape=jax.ShapeDtypeStruct((1, B), jnp.float32),
    )(encT, W1t, b1, W2t, b2, W3t, b3)


def kernel(x, emb_tables, W1, b1, W2, b2, W3, b3):
    tableT = emb_tables.transpose(0, 2, 1)
    xt3 = x.T.reshape(NV, B // GROUP, GROUP)
    encT = _sc_gather(tableT, xt3).reshape(IN_DIM, B)
    outT = _tc_mlp_t(
        encT,
        W1.T, b1.reshape(H1, 1),
        W2.T, b2.reshape(H2, 1),
        W3.T, b3.reshape(1, 1),
    )
    return outT.reshape(B, 1)
